# Initial kernel scaffold; baseline (speedup 1.0000x reference)
import jax
import jax.numpy as jnp
from jax.experimental import pallas as pl


def kernel(x, edge_index, edge_label_index, W1_l, b1, W1_r, W2_l, b2, W2_r, Wd1, bd1, Wd2, bd2):
    raise NotImplementedError("write your pallas kernel here")


# speedup vs baseline: 2.4324x; 2.4324x over previous
import functools

import jax
import jax.numpy as jnp
from jax import lax
from jax.experimental import pallas as pl
from jax.experimental.pallas import tpu as pltpu
from jax.experimental.pallas import tpu_sc as plsc

N = 10000
H = 128
E = 320000
EL = 200000

NW = 32
CH = 128
NPAD = 10112
RPT = NPAD // 16
EPAD = 79 * NW * CH
ELPAD = 50 * NW * CH

BN = 1000


def _dot_t(a, b):
    return lax.dot_general(a, b, (((1,), (1,)), ((), ())),
                           preferred_element_type=jnp.float32)


def _enc_body(x_ref, w_ref, o_ref):
    o_ref[...] = _dot_t(x_ref[...], w_ref[...])


_enc = pl.pallas_call(
    _enc_body,
    grid=(N // BN,),
    in_specs=[pl.BlockSpec((BN, H), lambda i: (i, 0)),
              pl.BlockSpec((H, H), lambda i: (0, 0))],
    out_specs=pl.BlockSpec((BN, H), lambda i: (i, 0)),
    out_shape=jax.ShapeDtypeStruct((N, H), jnp.float32),
)


def _mid_body(aa_ref, ab_ref, deg_ref, x_ref, w1r_ref, b1_ref, w2l_ref,
              w2r_ref, b2_ref, t2_ref, r2_ref):
    agg = aa_ref[...] + ab_ref[...]
    inv = 1.0 / jnp.maximum(deg_ref[...], 1.0)
    h = jnp.maximum(agg * inv + _dot_t(x_ref[...], w1r_ref[...])
                    + b1_ref[...], 0.0)
    t2_ref[...] = _dot_t(h, w2l_ref[...])
    r2_ref[...] = _dot_t(h, w2r_ref[...]) + b2_ref[...]


_mid = pl.pallas_call(
    _mid_body,
    grid=(N // BN,),
    in_specs=[pl.BlockSpec((BN, H), lambda i: (i, 0)),
              pl.BlockSpec((BN, H), lambda i: (i, 0)),
              pl.BlockSpec((BN, 1), lambda i: (i, 0)),
              pl.BlockSpec((BN, H), lambda i: (i, 0)),
              pl.BlockSpec((H, H), lambda i: (0, 0)),
              pl.BlockSpec((1, H), lambda i: (0, 0)),
              pl.BlockSpec((H, H), lambda i: (0, 0)),
              pl.BlockSpec((H, H), lambda i: (0, 0)),
              pl.BlockSpec((1, H), lambda i: (0, 0))],
    out_specs=[pl.BlockSpec((BN, H), lambda i: (i, 0)),
               pl.BlockSpec((BN, H), lambda i: (i, 0))],
    out_shape=[jax.ShapeDtypeStruct((N, H), jnp.float32)] * 2,
)


def _dec_prep_body(aa_ref, ab_ref, deg_ref, r2_ref, wdl_ref, wdr_ref,
                   bd1_ref, u_ref, v_ref):
    inv = 1.0 / jnp.maximum(deg_ref[...], 1.0)
    z = (aa_ref[...] + ab_ref[...]) * inv + r2_ref[...]
    u_ref[...] = _dot_t(z, wdl_ref[...]) + bd1_ref[...]
    v_ref[...] = _dot_t(z, wdr_ref[...])


_dec_prep = pl.pallas_call(
    _dec_prep_body,
    grid=(N // BN,),
    in_specs=[pl.BlockSpec((BN, H), lambda i: (i, 0)),
              pl.BlockSpec((BN, H), lambda i: (i, 0)),
              pl.BlockSpec((BN, 1), lambda i: (i, 0)),
              pl.BlockSpec((BN, H), lambda i: (i, 0)),
              pl.BlockSpec((H, H), lambda i: (0, 0)),
              pl.BlockSpec((H, H), lambda i: (0, 0)),
              pl.BlockSpec((1, H), lambda i: (0, 0))],
    out_specs=[pl.BlockSpec((BN, H), lambda i: (i, 0)),
               pl.BlockSpec((BN, H), lambda i: (i, 0))],
    out_shape=[jax.ShapeDtypeStruct((N, H), jnp.float32)] * 2,
)


_BF = 1024


def _fin_body(p_ref, bd2_ref, o_ref):
    s = jnp.sum(p_ref[...], axis=1)
    o_ref[...] = s.reshape(_BF // H, H) + bd2_ref[...]


_fin = pl.pallas_call(
    _fin_body,
    grid=(ELPAD // _BF,),
    in_specs=[pl.BlockSpec((_BF, 16), lambda i: (i, 0)),
              pl.BlockSpec((1, H), lambda i: (0, 0))],
    out_specs=pl.BlockSpec((_BF // H, H), lambda i: (i, 0)),
    out_shape=jax.ShapeDtypeStruct((ELPAD // H, H), jnp.float32),
)


_EW = EPAD // NW
_NCHUNKS = _EW // CH
_sc_mesh = plsc.VectorSubcoreMesh(core_axis_name="c", subcore_axis_name="s")


@functools.partial(
    pl.kernel,
    out_type=jax.ShapeDtypeStruct((2 * NPAD, H), jnp.float32),
    mesh=_sc_mesh,
    scratch_types=[
        pltpu.VMEM((CH,), jnp.int32),
        pltpu.VMEM((CH,), jnp.int32),
        pltpu.VMEM((CH, H), jnp.float32),
        pltpu.VMEM_SHARED((NPAD, H), jnp.float32),
        pltpu.SemaphoreType.DMA,
    ],
)
def _scatter_h(t_hbm, src_hbm, dst_hbm, zero_hbm,
               out_hbm, sidx, didx, rows, acc, sem):
    cid = lax.axis_index("c")
    sid = lax.axis_index("s")
    wid = sid * 2 + cid
    r0 = sid * RPT
    pltpu.sync_copy(zero_hbm.at[pl.ds(r0, RPT)], acc.at[pl.ds(r0, RPT)])
    plsc.subcore_barrier()

    def body(g, carry):
        base = wid * _EW + g * CH
        pltpu.sync_copy(src_hbm.at[pl.ds(base, CH)], sidx)
        pltpu.sync_copy(dst_hbm.at[pl.ds(base, CH)], didx)
        pltpu.async_copy(t_hbm.at[sidx], rows, sem).wait()
        pltpu.sync_copy(rows, acc.at[didx], add=True)
        return carry

    lax.fori_loop(0, _NCHUNKS, body, 0)
    plsc.subcore_barrier()
    pltpu.sync_copy(acc.at[pl.ds(r0, RPT)],
                    out_hbm.at[pl.ds(cid * NPAD + r0, RPT)])


@functools.partial(
    pl.kernel,
    out_type=jax.ShapeDtypeStruct((2 * NPAD, H), jnp.float32),
    mesh=_sc_mesh,
    scratch_types=[
        pltpu.VMEM((CH,), jnp.int32),
        pltpu.VMEM((CH, H), jnp.float32),
        pltpu.VMEM_SHARED((NPAD, H), jnp.float32),
    ],
)
def _deg_kernel(dst_hbm, zero_hbm, ones_hbm, out_hbm, didx, onesv, acc):
    cid = lax.axis_index("c")
    sid = lax.axis_index("s")
    wid = sid * 2 + cid
    r0 = sid * RPT
    pltpu.sync_copy(zero_hbm.at[pl.ds(r0, RPT)], acc.at[pl.ds(r0, RPT)])
    pltpu.sync_copy(ones_hbm, onesv)
    plsc.subcore_barrier()

    def body(g, carry):
        base = wid * _EW + g * CH
        pltpu.sync_copy(dst_hbm.at[pl.ds(base, CH)], didx)
        pltpu.sync_copy(onesv, acc.at[didx], add=True)
        return carry

    lax.fori_loop(0, _NCHUNKS, body, 0)
    plsc.subcore_barrier()
    pltpu.sync_copy(acc.at[pl.ds(r0, RPT)],
                    out_hbm.at[pl.ds(cid * NPAD + r0, RPT)])


_elw = ELPAD // NW
_elchunks = _elw // CH
_dec_mesh = plsc.VectorSubcoreMesh(core_axis_name="c", subcore_axis_name="s")


@functools.partial(
    pl.kernel,
    out_type=jax.ShapeDtypeStruct((ELPAD, 16), jnp.float32),
    mesh=_dec_mesh,
    scratch_types=[
        pltpu.VMEM((CH,), jnp.int32),
        pltpu.VMEM((CH,), jnp.int32),
        pltpu.VMEM((CH, H), jnp.float32),
        pltpu.VMEM((CH, H), jnp.float32),
        pltpu.VMEM((CH, 16), jnp.float32),
        pltpu.VMEM((H,), jnp.float32),
        pltpu.SemaphoreType.DMA,
    ],
)
def _decode(u_hbm, v_hbm, i0_hbm, i1_hbm, w_hbm, out_hbm,
            i0v, i1v, ur, vr, pr, wv, sem):
    cid = lax.axis_index("c")
    sid = lax.axis_index("s")
    wid = sid * 2 + cid
    pltpu.sync_copy(w_hbm, wv)

    def chunk(g, carry):
        base = wid * _elw + g * CH
        pltpu.sync_copy(i0_hbm.at[pl.ds(base, CH)], i0v)
        pltpu.sync_copy(i1_hbm.at[pl.ds(base, CH)], i1v)
        pltpu.async_copy(u_hbm.at[i0v], ur, sem).wait()
        pltpu.async_copy(v_hbm.at[i1v], vr, sem).wait()

        def edge(e, c2):
            acc = jnp.zeros((16,), jnp.float32)
            for j in range(H // 16):
                u = ur[e, pl.ds(j * 16, 16)]
                v = vr[e, pl.ds(j * 16, 16)]
                w = wv[pl.ds(j * 16, 16)]
                acc = acc + jnp.maximum(u + v, 0.0) * w
            pr[e, :] = acc
            return c2

        lax.fori_loop(0, CH, edge, 0)
        pltpu.sync_copy(pr, out_hbm.at[pl.ds(base, CH)])
        return carry

    lax.fori_loop(0, _elchunks, chunk, 0)


def kernel(x, edge_index, edge_label_index, W1_l, b1, W1_r, W2_l, b2, W2_r,
           Wd1, bd1, Wd2, bd2):
    src = edge_index[0].astype(jnp.int32)
    dst = edge_index[1].astype(jnp.int32)
    srcp = jnp.concatenate([src, jnp.zeros((EPAD - E,), jnp.int32)])
    dstp = jnp.concatenate([dst, jnp.full((EPAD - E,), N, jnp.int32)])
    i0 = jnp.concatenate([edge_label_index[0].astype(jnp.int32),
                          jnp.zeros((ELPAD - EL,), jnp.int32)])
    i1 = jnp.concatenate([edge_label_index[1].astype(jnp.int32),
                          jnp.zeros((ELPAD - EL,), jnp.int32)])
    zeros = jnp.zeros((NPAD, H), jnp.float32)
    onesb = jnp.ones((CH, H), jnp.float32)
    b1r = b1.reshape(1, H)
    b2r = b2.reshape(1, H)
    bd1r = bd1.reshape(1, H)
    bd2r = jnp.broadcast_to(bd2.reshape(1, 1), (1, H))

    t1 = _enc(x, W1_l)
    degs = _deg_kernel(dstp, zeros, onesb)
    s1 = _scatter_h(t1, srcp, dstp, zeros)
    deg = (degs[:NPAD] + degs[NPAD:])[:N, :1]
    t2, r2b = _mid(s1[:N], s1[NPAD:NPAD + N], deg, x, W1_r, b1r, W2_l,
                   W2_r, b2r)
    s2 = _scatter_h(t2, srcp, dstp, zeros)
    u, v = _dec_prep(s2[:N], s2[NPAD:NPAD + N], deg, r2b,
                     Wd1[:, :H], Wd1[:, H:], bd1r)
    partial = _decode(u, v, i0, i1, Wd2[0])
    out2d = _fin(partial, bd2r)
    return out2d.reshape(-1)[:EL]


# speedup vs baseline: 2.8909x; 1.1885x over previous
import functools

import jax
import jax.numpy as jnp
from jax import lax
from jax.experimental import pallas as pl
from jax.experimental.pallas import tpu as pltpu
from jax.experimental.pallas import tpu_sc as plsc

N = 10000
H = 128
E = 320000
EL = 200000

NW = 32
CH = 128
NPAD = 10112
RPT = NPAD // 16
EPAD = 80 * NW * CH
ELPAD = 50 * NW * CH

BN = 1000


def _dot_t(a, b):
    return lax.dot_general(a, b, (((1,), (1,)), ((), ())),
                           preferred_element_type=jnp.float32)


def _enc_body(x_ref, w_ref, o_ref):
    o_ref[...] = _dot_t(x_ref[...], w_ref[...])


_enc = pl.pallas_call(
    _enc_body,
    grid=(N // BN,),
    in_specs=[pl.BlockSpec((BN, H), lambda i: (i, 0)),
              pl.BlockSpec((H, H), lambda i: (0, 0))],
    out_specs=pl.BlockSpec((BN, H), lambda i: (i, 0)),
    out_shape=jax.ShapeDtypeStruct((N, H), jnp.float32),
)


def _mid_body(aa_ref, ab_ref, deg_ref, x_ref, w1r_ref, b1_ref, w2l_ref,
              w2r_ref, b2_ref, t2_ref, r2_ref):
    agg = aa_ref[...] + ab_ref[...]
    inv = 1.0 / jnp.maximum(deg_ref[...], 1.0)
    h = jnp.maximum(agg * inv + _dot_t(x_ref[...], w1r_ref[...])
                    + b1_ref[...], 0.0)
    t2_ref[...] = _dot_t(h, w2l_ref[...])
    r2_ref[...] = _dot_t(h, w2r_ref[...]) + b2_ref[...]


_mid = pl.pallas_call(
    _mid_body,
    grid=(N // BN,),
    in_specs=[pl.BlockSpec((BN, H), lambda i: (i, 0)),
              pl.BlockSpec((BN, H), lambda i: (i, 0)),
              pl.BlockSpec((BN, 1), lambda i: (i, 0)),
              pl.BlockSpec((BN, H), lambda i: (i, 0)),
              pl.BlockSpec((H, H), lambda i: (0, 0)),
              pl.BlockSpec((1, H), lambda i: (0, 0)),
              pl.BlockSpec((H, H), lambda i: (0, 0)),
              pl.BlockSpec((H, H), lambda i: (0, 0)),
              pl.BlockSpec((1, H), lambda i: (0, 0))],
    out_specs=[pl.BlockSpec((BN, H), lambda i: (i, 0)),
               pl.BlockSpec((BN, H), lambda i: (i, 0))],
    out_shape=[jax.ShapeDtypeStruct((N, H), jnp.float32)] * 2,
)


def _dec_prep_body(aa_ref, ab_ref, deg_ref, r2_ref, wdl_ref, wdr_ref,
                   bd1_ref, u_ref, v_ref):
    inv = 1.0 / jnp.maximum(deg_ref[...], 1.0)
    z = (aa_ref[...] + ab_ref[...]) * inv + r2_ref[...]
    u_ref[...] = _dot_t(z, wdl_ref[...]) + bd1_ref[...]
    v_ref[...] = _dot_t(z, wdr_ref[...])


_dec_prep = pl.pallas_call(
    _dec_prep_body,
    grid=(N // BN,),
    in_specs=[pl.BlockSpec((BN, H), lambda i: (i, 0)),
              pl.BlockSpec((BN, H), lambda i: (i, 0)),
              pl.BlockSpec((BN, 1), lambda i: (i, 0)),
              pl.BlockSpec((BN, H), lambda i: (i, 0)),
              pl.BlockSpec((H, H), lambda i: (0, 0)),
              pl.BlockSpec((H, H), lambda i: (0, 0)),
              pl.BlockSpec((1, H), lambda i: (0, 0))],
    out_specs=[pl.BlockSpec((BN, H), lambda i: (i, 0)),
               pl.BlockSpec((BN, H), lambda i: (i, 0))],
    out_shape=[jax.ShapeDtypeStruct((N, H), jnp.float32)] * 2,
)


_BF = 1024


def _fin_body(p_ref, bd2_ref, o_ref):
    s = jnp.sum(p_ref[...], axis=1)
    o_ref[...] = s.reshape(_BF // H, H) + bd2_ref[...]


_fin = pl.pallas_call(
    _fin_body,
    grid=(ELPAD // _BF,),
    in_specs=[pl.BlockSpec((_BF, 16), lambda i: (i, 0)),
              pl.BlockSpec((1, H), lambda i: (0, 0))],
    out_specs=pl.BlockSpec((_BF // H, H), lambda i: (i, 0)),
    out_shape=jax.ShapeDtypeStruct((ELPAD // H, H), jnp.float32),
)


_EW = EPAD // NW
_NCHUNKS = _EW // CH
_sc_mesh = plsc.VectorSubcoreMesh(core_axis_name="c", subcore_axis_name="s")


@functools.partial(
    pl.kernel,
    out_type=jax.ShapeDtypeStruct((2 * NPAD, H), jnp.float32),
    mesh=_sc_mesh,
    scratch_types=[
        pltpu.VMEM((2, CH), jnp.int32),
        pltpu.VMEM((2, CH), jnp.int32),
        pltpu.VMEM((2, CH, H), jnp.float32),
        pltpu.VMEM_SHARED((NPAD, H), jnp.float32),
        pltpu.SemaphoreType.DMA,
        pltpu.SemaphoreType.DMA,
    ],
)
def _scatter_h(t_hbm, src_hbm, dst_hbm, zero_hbm,
               out_hbm, sidx, didx, rows, acc, sem0, sem1):
    cid = lax.axis_index("c")
    sid = lax.axis_index("s")
    wid = sid * 2 + cid
    r0 = sid * RPT
    pltpu.sync_copy(zero_hbm.at[pl.ds(r0, RPT)], acc.at[pl.ds(r0, RPT)])
    plsc.subcore_barrier()

    sems = (sem0, sem1)

    def loadfire(g, b):
        base = wid * _EW + g * CH
        pltpu.sync_copy(src_hbm.at[pl.ds(base, CH)], sidx.at[b])
        pltpu.sync_copy(dst_hbm.at[pl.ds(base, CH)], didx.at[b])
        pltpu.async_copy(t_hbm.at[sidx.at[b]], rows.at[b], sems[b])

    def drain(b):
        pltpu.make_async_copy(t_hbm.at[pl.ds(0, CH)], rows.at[b],
                              sems[b]).wait()

    def scat(b):
        pltpu.sync_copy(rows.at[b], acc.at[didx.at[b]], add=True)

    loadfire(0, 0)

    def outer(gg, carry):
        g0 = 2 * gg
        loadfire(g0 + 1, 1)
        drain(0)
        scat(0)

        @pl.when(gg < _NCHUNKS // 2 - 1)
        def _():
            loadfire(g0 + 2, 0)

        drain(1)
        scat(1)
        return carry

    lax.fori_loop(0, _NCHUNKS // 2, outer, 0)
    plsc.subcore_barrier()
    pltpu.sync_copy(acc.at[pl.ds(r0, RPT)],
                    out_hbm.at[pl.ds(cid * NPAD + r0, RPT)])


@functools.partial(
    pl.kernel,
    out_type=jax.ShapeDtypeStruct((2 * NPAD, H), jnp.float32),
    mesh=_sc_mesh,
    scratch_types=[
        pltpu.VMEM((CH,), jnp.int32),
        pltpu.VMEM((CH, H), jnp.float32),
        pltpu.VMEM_SHARED((NPAD, H), jnp.float32),
    ],
)
def _deg_kernel(dst_hbm, zero_hbm, ones_hbm, out_hbm, didx, onesv, acc):
    cid = lax.axis_index("c")
    sid = lax.axis_index("s")
    wid = sid * 2 + cid
    r0 = sid * RPT
    pltpu.sync_copy(zero_hbm.at[pl.ds(r0, RPT)], acc.at[pl.ds(r0, RPT)])
    pltpu.sync_copy(ones_hbm, onesv)
    plsc.subcore_barrier()

    def body(g, carry):
        base = wid * _EW + g * CH
        pltpu.sync_copy(dst_hbm.at[pl.ds(base, CH)], didx)
        pltpu.sync_copy(onesv, acc.at[didx], add=True)
        return carry

    lax.fori_loop(0, _NCHUNKS, body, 0)
    plsc.subcore_barrier()
    pltpu.sync_copy(acc.at[pl.ds(r0, RPT)],
                    out_hbm.at[pl.ds(cid * NPAD + r0, RPT)])


_elw = ELPAD // NW
_elchunks = _elw // CH
_dec_mesh = plsc.VectorSubcoreMesh(core_axis_name="c", subcore_axis_name="s")


@functools.partial(
    pl.kernel,
    out_type=jax.ShapeDtypeStruct((ELPAD, 16), jnp.float32),
    mesh=_dec_mesh,
    scratch_types=[
        pltpu.VMEM((2, CH), jnp.int32),
        pltpu.VMEM((2, CH), jnp.int32),
        pltpu.VMEM((2, CH, H), jnp.float32),
        pltpu.VMEM((2, CH, H), jnp.float32),
        pltpu.VMEM((CH, 16), jnp.float32),
        pltpu.VMEM((H,), jnp.float32),
        pltpu.SemaphoreType.DMA,
        pltpu.SemaphoreType.DMA,
    ],
)
def _decode(u_hbm, v_hbm, i0_hbm, i1_hbm, w_hbm, out_hbm,
            i0v, i1v, ur, vr, pr, wv, sem0, sem1):
    cid = lax.axis_index("c")
    sid = lax.axis_index("s")
    wid = sid * 2 + cid
    pltpu.sync_copy(w_hbm, wv)
    wr = [wv[pl.ds(j * 16, 16)] for j in range(H // 16)]
    sems = (sem0, sem1)

    def loadfire(g, b):
        base = wid * _elw + g * CH
        pltpu.sync_copy(i0_hbm.at[pl.ds(base, CH)], i0v.at[b])
        pltpu.sync_copy(i1_hbm.at[pl.ds(base, CH)], i1v.at[b])
        pltpu.async_copy(u_hbm.at[i0v.at[b]], ur.at[b], sems[b])
        pltpu.async_copy(v_hbm.at[i1v.at[b]], vr.at[b], sems[b])

    def drain(b):
        pltpu.make_async_copy(u_hbm.at[pl.ds(0, CH)], ur.at[b],
                              sems[b]).wait()
        pltpu.make_async_copy(v_hbm.at[pl.ds(0, CH)], vr.at[b],
                              sems[b]).wait()

    def compute(g, b):
        def edge(e, c2):
            acc = jnp.zeros((16,), jnp.float32)
            for j in range(H // 16):
                u = ur[b, e, pl.ds(j * 16, 16)]
                v = vr[b, e, pl.ds(j * 16, 16)]
                acc = acc + jnp.maximum(u + v, 0.0) * wr[j]
            pr[e, :] = acc
            return c2

        lax.fori_loop(0, CH, edge, 0)
        base = wid * _elw + g * CH
        pltpu.sync_copy(pr, out_hbm.at[pl.ds(base, CH)])

    loadfire(0, 0)

    def outer(gg, carry):
        g0 = 2 * gg
        loadfire(g0 + 1, 1)
        drain(0)
        compute(g0, 0)

        @pl.when(gg < _elchunks // 2 - 1)
        def _():
            loadfire(g0 + 2, 0)

        drain(1)
        compute(g0 + 1, 1)
        return carry

    lax.fori_loop(0, _elchunks // 2, outer, 0)


def kernel(x, edge_index, edge_label_index, W1_l, b1, W1_r, W2_l, b2, W2_r,
           Wd1, bd1, Wd2, bd2):
    src = edge_index[0].astype(jnp.int32)
    dst = edge_index[1].astype(jnp.int32)
    srcp = jnp.concatenate([src, jnp.zeros((EPAD - E,), jnp.int32)])
    dstp = jnp.concatenate([dst, jnp.full((EPAD - E,), N, jnp.int32)])
    i0 = jnp.concatenate([edge_label_index[0].astype(jnp.int32),
                          jnp.zeros((ELPAD - EL,), jnp.int32)])
    i1 = jnp.concatenate([edge_label_index[1].astype(jnp.int32),
                          jnp.zeros((ELPAD - EL,), jnp.int32)])
    zeros = jnp.zeros((NPAD, H), jnp.float32)
    onesb = jnp.ones((CH, H), jnp.float32)
    b1r = b1.reshape(1, H)
    b2r = b2.reshape(1, H)
    bd1r = bd1.reshape(1, H)
    bd2r = jnp.broadcast_to(bd2.reshape(1, 1), (1, H))

    t1 = _enc(x, W1_l)
    degs = _deg_kernel(dstp, zeros, onesb)
    s1 = _scatter_h(t1, srcp, dstp, zeros)
    deg = (degs[:NPAD] + degs[NPAD:])[:N, :1]
    t2, r2b = _mid(s1[:N], s1[NPAD:NPAD + N], deg, x, W1_r, b1r, W2_l,
                   W2_r, b2r)
    s2 = _scatter_h(t2, srcp, dstp, zeros)
    u, v = _dec_prep(s2[:N], s2[NPAD:NPAD + N], deg, r2b,
                     Wd1[:, :H], Wd1[:, H:], bd1r)
    partial = _decode(u, v, i0, i1, Wd2[0])
    out2d = _fin(partial, bd2r)
    return out2d.reshape(-1)[:EL]
